# baseline (speedup 1.0000x reference)
import functools

import jax
import jax.numpy as jnp
from jax import lax
from jax.experimental import pallas as pl
from jax.experimental.pallas import tpu as pltpu
from jax.experimental.pallas import tpu_sc as plsc

N_NODES = 10000
N_EDGES = 320000
IN_C = 128
HID_C = 256
OUT_C = 256

NC = 2
NS = 16
EPB = 128

NROWS = N_NODES
ACC_ROWS = 10112
EPAD = 323584
MSG_NBLK = EPAD // NS // EPB
MSG_NCHUNK = 2
MSG_CBLK = MSG_NBLK // MSG_NCHUNK
DEG_NBLK = EPAD // (NC * NS) // EPB
ZPS = ACC_ROWS // NS
WPS = 632
WLAST = NROWS - (NS - 1) * WPS
HALF = 128

@functools.cache
def _mesh():
    return plsc.VectorSubcoreMesh(
        core_axis_name="c", subcore_axis_name="s",
        num_cores=NC, num_subcores=NS)


def _deg_body(dst_hbm, ones_hbm, zrows_hbm, deg_hbm, dst_v, ones_v, deg_sh,
              sem):
    c = lax.axis_index("c")
    s = lax.axis_index("s")
    wid = c * NS + s
    pltpu.sync_copy(zrows_hbm, deg_sh.at[pl.ds(s * ZPS, ZPS)])
    pltpu.sync_copy(ones_hbm, ones_v)
    pltpu.sync_copy(dst_hbm.at[wid], dst_v)
    plsc.subcore_barrier()

    def body(j, carry):
        pltpu.sync_copy(ones_v, deg_sh.at[dst_v.at[j]], add=True)
        return carry

    lax.fori_loop(0, DEG_NBLK, body, 0)
    plsc.subcore_barrier()

    @pl.when(s < NS - 1)
    def _():
        pltpu.sync_copy(deg_sh.at[pl.ds(s * WPS, WPS)],
                        deg_hbm.at[pl.ds(c * NROWS + s * WPS, WPS)])

    @pl.when(s == NS - 1)
    def _():
        pltpu.sync_copy(deg_sh.at[pl.ds((NS - 1) * WPS, WLAST)],
                        deg_hbm.at[pl.ds(c * NROWS + (NS - 1) * WPS, WLAST)])


@functools.cache
def _deg_kernel():
    return pl.kernel(
        _deg_body,
        out_type=jax.ShapeDtypeStruct((NC * NROWS, HALF), jnp.float32),
        mesh=_mesh(),
        scratch_types=[
            pltpu.VMEM((DEG_NBLK, EPB), jnp.int32),
            pltpu.VMEM((EPB, HALF), jnp.float32),
            pltpu.VMEM_SHARED((ACC_ROWS, HALF), jnp.float32),
            pltpu.SemaphoreType.DMA,
        ],
    )


def _msg_half(g_hbm, out_hbm, src_hbm, dst_hbm, src_v, dst_v, rows_v,
              acc_sh, sem, s):
    def chunk(k, carry):
        pltpu.sync_copy(src_hbm.at[s, k], src_v)
        pltpu.sync_copy(dst_hbm.at[s, k], dst_v)

        def body(j, c2):
            pltpu.async_copy(g_hbm.at[src_v.at[j]], rows_v, sem).wait()
            pltpu.sync_copy(rows_v, acc_sh.at[dst_v.at[j]], add=True)
            return c2

        lax.fori_loop(0, MSG_CBLK, body, 0)
        return carry

    lax.fori_loop(0, MSG_NCHUNK, chunk, 0)
    plsc.subcore_barrier()

    @pl.when(s < NS - 1)
    def _():
        pltpu.sync_copy(acc_sh.at[pl.ds(s * WPS, WPS)],
                        out_hbm.at[pl.ds(s * WPS, WPS)])

    @pl.when(s == NS - 1)
    def _():
        pltpu.sync_copy(acc_sh.at[pl.ds((NS - 1) * WPS, WLAST)],
                        out_hbm.at[pl.ds((NS - 1) * WPS, WLAST)])


def _msg_body(src_hbm, dst_hbm, ga_hbm, gb_hbm, zrows_hbm,
              acca_hbm, accb_hbm, src_v, dst_v, rows_v, acc_sh, sem):
    c = lax.axis_index("c")
    s = lax.axis_index("s")
    pltpu.sync_copy(zrows_hbm, acc_sh.at[pl.ds(s * ZPS, ZPS)])
    plsc.subcore_barrier()

    @pl.when(c == 0)
    def _():
        _msg_half(ga_hbm, acca_hbm, src_hbm, dst_hbm, src_v, dst_v, rows_v,
                  acc_sh, sem, s)

    @pl.when(c == 1)
    def _():
        _msg_half(gb_hbm, accb_hbm, src_hbm, dst_hbm, src_v, dst_v, rows_v,
                  acc_sh, sem, s)


@functools.cache
def _msg_kernel():
    return pl.kernel(
        _msg_body,
        out_type=[jax.ShapeDtypeStruct((NROWS, HALF), jnp.float32),
                  jax.ShapeDtypeStruct((NROWS, HALF), jnp.float32)],
        mesh=_mesh(),
        scratch_types=[
            pltpu.VMEM((MSG_CBLK, EPB), jnp.int32),
            pltpu.VMEM((MSG_CBLK, EPB), jnp.int32),
            pltpu.VMEM((EPB, HALF), jnp.float32),
            pltpu.VMEM_SHARED((ACC_ROWS, HALF), jnp.float32),
            pltpu.SemaphoreType.DMA,
        ],
    )


_RB = 1000


def _dinv_of(deg_ref):
    return lax.rsqrt(deg_ref[0, :, :1] + deg_ref[1, :, :1] + 1.0)


def _tc_first_body(x_ref, w_ref, deg_ref, ga_ref, gb_ref):
    dinv = _dinv_of(deg_ref)
    h = jnp.dot(x_ref[...], w_ref[...], preferred_element_type=jnp.float32)
    g = h * dinv
    ga_ref[...] = g[:, :HALF]
    gb_ref[...] = g[:, HALF:]


def _tc_mid_body(acca_ref, accb_ref, ga_ref, gb_ref, deg_ref, w_ref, b_ref,
                 oa_ref, ob_ref):
    dinv = _dinv_of(deg_ref)
    pre = jnp.concatenate(
        [acca_ref[...] + ga_ref[...], accb_ref[...] + gb_ref[...]], axis=1)
    act = jnp.maximum(pre * dinv + b_ref[...], 0.0)
    h = jnp.dot(act, w_ref[...], preferred_element_type=jnp.float32)
    g = h * dinv
    oa_ref[...] = g[:, :HALF]
    ob_ref[...] = g[:, HALF:]


def _tc_last_body(acca_ref, accb_ref, ga_ref, gb_ref, deg_ref, b_ref, o_ref):
    dinv = _dinv_of(deg_ref)
    pre = jnp.concatenate(
        [acca_ref[...] + ga_ref[...], accb_ref[...] + gb_ref[...]], axis=1)
    o_ref[...] = pre * dinv + b_ref[...]


def _half_spec():
    return pl.BlockSpec((_RB, HALF), lambda i: (i, 0))


def _deg_spec():
    return pl.BlockSpec((2, _RB, HALF), lambda i: (0, i, 0))


def _full_spec(cols):
    return pl.BlockSpec((_RB, cols), lambda i: (i, 0))


def _const_spec(r, c):
    return pl.BlockSpec((r, c), lambda i: (0, 0))


def _tc_first(x, w, deg):
    return pl.pallas_call(
        _tc_first_body,
        grid=(NROWS // _RB,),
        in_specs=[_full_spec(IN_C), _const_spec(IN_C, HID_C), _deg_spec()],
        out_specs=[_half_spec(), _half_spec()],
        out_shape=[jax.ShapeDtypeStruct((NROWS, HALF), jnp.float32)] * 2,
    )(x, w, deg)


def _tc_mid(acca, accb, ga, gb, deg, w, b):
    return pl.pallas_call(
        _tc_mid_body,
        grid=(NROWS // _RB,),
        in_specs=[_half_spec(), _half_spec(), _half_spec(), _half_spec(),
                  _deg_spec(), _const_spec(HID_C, HID_C), _const_spec(1, HID_C)],
        out_specs=[_half_spec(), _half_spec()],
        out_shape=[jax.ShapeDtypeStruct((NROWS, HALF), jnp.float32)] * 2,
    )(acca, accb, ga, gb, deg, w, b)


def _tc_last(acca, accb, ga, gb, deg, b):
    return pl.pallas_call(
        _tc_last_body,
        grid=(NROWS // _RB,),
        in_specs=[_half_spec(), _half_spec(), _half_spec(), _half_spec(),
                  _deg_spec(), _const_spec(1, OUT_C)],
        out_specs=_full_spec(OUT_C),
        out_shape=jax.ShapeDtypeStruct((NROWS, OUT_C), jnp.float32),
    )(acca, accb, ga, gb, deg, b)


def kernel(x, edge_index, W1, b1, W2, b2, W3, b3):
    ei = edge_index.astype(jnp.int32)
    pad = EPAD - N_EDGES
    src_p = jnp.concatenate([ei[0], jnp.zeros((pad,), jnp.int32)])
    dst_p = jnp.concatenate([ei[1], jnp.full((pad,), N_NODES, jnp.int32)])
    src16 = src_p.reshape(NS, MSG_NCHUNK, MSG_CBLK, EPB)
    dst16 = dst_p.reshape(NS, MSG_NCHUNK, MSG_CBLK, EPB)
    dst32 = dst_p.reshape(NC * NS, DEG_NBLK, EPB)

    ones = jnp.ones((EPB, HALF), jnp.float32)
    zrows = jnp.zeros((ZPS, HALF), jnp.float32)

    deg = _deg_kernel()(dst32, ones, zrows).reshape(NC, NROWS, HALF)

    msg = _msg_kernel()
    g1a, g1b = _tc_first(x, W1, deg)
    a1a, a1b = msg(src16, dst16, g1a, g1b, zrows)

    g2a, g2b = _tc_mid(a1a, a1b, g1a, g1b, deg, W2, b1.reshape(1, HID_C))
    a2a, a2b = msg(src16, dst16, g2a, g2b, zrows)

    g3a, g3b = _tc_mid(a2a, a2b, g2a, g2b, deg, W3, b2.reshape(1, HID_C))
    a3a, a3b = msg(src16, dst16, g3a, g3b, zrows)

    return _tc_last(a3a, a3b, g3a, g3b, deg, b3.reshape(1, OUT_C))


# speedup vs baseline: 1.0312x; 1.0312x over previous
import functools

import jax
import jax.numpy as jnp
from jax import lax
from jax.experimental import pallas as pl
from jax.experimental.pallas import tpu as pltpu
from jax.experimental.pallas import tpu_sc as plsc

N_NODES = 10000
N_EDGES = 320000
IN_C = 128
HID_C = 256
OUT_C = 256

NC = 2
NS = 16
EPB = 128

NPAD = 10240
EPAD = 323584
MSG_NBLK = EPAD // NS // EPB
MSG_NCHUNK = 2
MSG_CBLK = MSG_NBLK // MSG_NCHUNK
DEG_NBLK = EPAD // (NC * NS) // EPB
RPS = NPAD // NS
HALF = 128

@functools.cache
def _mesh():
    return plsc.VectorSubcoreMesh(
        core_axis_name="c", subcore_axis_name="s",
        num_cores=NC, num_subcores=NS)


def _deg_body(dst_hbm, ones_hbm, zrows_hbm, deg_hbm, dst_v, ones_v, deg_sh,
              sem):
    c = lax.axis_index("c")
    s = lax.axis_index("s")
    wid = c * NS + s
    pltpu.sync_copy(zrows_hbm, deg_sh.at[pl.ds(s * RPS, RPS)])
    pltpu.sync_copy(ones_hbm, ones_v)
    pltpu.sync_copy(dst_hbm.at[wid], dst_v)
    plsc.subcore_barrier()

    def body(j, carry):
        pltpu.sync_copy(ones_v, deg_sh.at[dst_v.at[j]], add=True)
        return carry

    lax.fori_loop(0, DEG_NBLK, body, 0)
    plsc.subcore_barrier()
    pltpu.sync_copy(deg_sh.at[pl.ds(s * RPS, RPS)],
                    deg_hbm.at[pl.ds(c * NPAD + s * RPS, RPS)])


@functools.cache
def _deg_kernel():
    return pl.kernel(
        _deg_body,
        out_type=jax.ShapeDtypeStruct((NC * NPAD, HALF), jnp.float32),
        mesh=_mesh(),
        scratch_types=[
            pltpu.VMEM((DEG_NBLK, EPB), jnp.int32),
            pltpu.VMEM((EPB, HALF), jnp.float32),
            pltpu.VMEM_SHARED((NPAD, HALF), jnp.float32),
            pltpu.SemaphoreType.DMA,
        ],
    )


def _msg_half(g_hbm, out_hbm, src_hbm, dst_hbm, src_v, dst_v, rows_v,
              acc_sh, sem, s):
    def chunk(k, carry):
        pltpu.sync_copy(src_hbm.at[s, k], src_v)
        pltpu.sync_copy(dst_hbm.at[s, k], dst_v)

        def body(j, c2):
            pltpu.async_copy(g_hbm.at[src_v.at[j]], rows_v, sem).wait()
            pltpu.sync_copy(rows_v, acc_sh.at[dst_v.at[j]], add=True)
            return c2

        lax.fori_loop(0, MSG_CBLK, body, 0)
        return carry

    lax.fori_loop(0, MSG_NCHUNK, chunk, 0)
    plsc.subcore_barrier()
    pltpu.sync_copy(acc_sh.at[pl.ds(s * RPS, RPS)],
                    out_hbm.at[pl.ds(s * RPS, RPS)])


def _msg_body(src_hbm, dst_hbm, ga_hbm, gb_hbm, zrows_hbm,
              acca_hbm, accb_hbm, src_v, dst_v, rows_v, acc_sh, sem):
    c = lax.axis_index("c")
    s = lax.axis_index("s")
    pltpu.sync_copy(zrows_hbm, acc_sh.at[pl.ds(s * RPS, RPS)])
    plsc.subcore_barrier()

    @pl.when(c == 0)
    def _():
        _msg_half(ga_hbm, acca_hbm, src_hbm, dst_hbm, src_v, dst_v, rows_v,
                  acc_sh, sem, s)

    @pl.when(c == 1)
    def _():
        _msg_half(gb_hbm, accb_hbm, src_hbm, dst_hbm, src_v, dst_v, rows_v,
                  acc_sh, sem, s)


@functools.cache
def _msg_kernel():
    return pl.kernel(
        _msg_body,
        out_type=[jax.ShapeDtypeStruct((NPAD, HALF), jnp.float32),
                  jax.ShapeDtypeStruct((NPAD, HALF), jnp.float32)],
        mesh=_mesh(),
        scratch_types=[
            pltpu.VMEM((MSG_CBLK, EPB), jnp.int32),
            pltpu.VMEM((MSG_CBLK, EPB), jnp.int32),
            pltpu.VMEM((EPB, HALF), jnp.float32),
            pltpu.VMEM_SHARED((NPAD, HALF), jnp.float32),
            pltpu.SemaphoreType.DMA,
        ],
    )


_RB = 1024


def _dinv_of(deg_ref):
    return lax.rsqrt(deg_ref[0, :, :1] + deg_ref[1, :, :1] + 1.0)


def _tc_first_body(x_ref, w_ref, deg_ref, ga_ref, gb_ref):
    dinv = _dinv_of(deg_ref)
    h = jnp.dot(x_ref[...], w_ref[...], preferred_element_type=jnp.float32)
    g = h * dinv
    ga_ref[...] = g[:, :HALF]
    gb_ref[...] = g[:, HALF:]


def _tc_mid_body(acca_ref, accb_ref, ga_ref, gb_ref, deg_ref, w_ref, b_ref,
                 oa_ref, ob_ref):
    dinv = _dinv_of(deg_ref)
    pre = jnp.concatenate(
        [acca_ref[...] + ga_ref[...], accb_ref[...] + gb_ref[...]], axis=1)
    act = jnp.maximum(pre * dinv + b_ref[...], 0.0)
    h = jnp.dot(act, w_ref[...], preferred_element_type=jnp.float32)
    g = h * dinv
    oa_ref[...] = g[:, :HALF]
    ob_ref[...] = g[:, HALF:]


def _tc_last_body(acca_ref, accb_ref, ga_ref, gb_ref, deg_ref, b_ref, o_ref):
    dinv = _dinv_of(deg_ref)
    pre = jnp.concatenate(
        [acca_ref[...] + ga_ref[...], accb_ref[...] + gb_ref[...]], axis=1)
    o_ref[...] = pre * dinv + b_ref[...]


def _half_spec():
    return pl.BlockSpec((_RB, HALF), lambda i: (i, 0))


def _deg_spec():
    return pl.BlockSpec((2, _RB, HALF), lambda i: (0, i, 0))


def _full_spec(cols):
    return pl.BlockSpec((_RB, cols), lambda i: (i, 0))


def _const_spec(r, c):
    return pl.BlockSpec((r, c), lambda i: (0, 0))


def _tc_first(x, w, deg):
    return pl.pallas_call(
        _tc_first_body,
        grid=(NPAD // _RB,),
        in_specs=[_full_spec(IN_C), _const_spec(IN_C, HID_C), _deg_spec()],
        out_specs=[_half_spec(), _half_spec()],
        out_shape=[jax.ShapeDtypeStruct((NPAD, HALF), jnp.float32)] * 2,
    )(x, w, deg)


def _tc_mid(acca, accb, ga, gb, deg, w, b):
    return pl.pallas_call(
        _tc_mid_body,
        grid=(NPAD // _RB,),
        in_specs=[_half_spec(), _half_spec(), _half_spec(), _half_spec(),
                  _deg_spec(), _const_spec(HID_C, HID_C), _const_spec(1, HID_C)],
        out_specs=[_half_spec(), _half_spec()],
        out_shape=[jax.ShapeDtypeStruct((NPAD, HALF), jnp.float32)] * 2,
    )(acca, accb, ga, gb, deg, w, b)


def _tc_last(acca, accb, ga, gb, deg, b):
    return pl.pallas_call(
        _tc_last_body,
        grid=(NPAD // _RB,),
        in_specs=[_half_spec(), _half_spec(), _half_spec(), _half_spec(),
                  _deg_spec(), _const_spec(1, OUT_C)],
        out_specs=_full_spec(OUT_C),
        out_shape=jax.ShapeDtypeStruct((NPAD, OUT_C), jnp.float32),
    )(acca, accb, ga, gb, deg, b)


def kernel(x, edge_index, W1, b1, W2, b2, W3, b3):
    ei = edge_index.astype(jnp.int32)
    pad = EPAD - N_EDGES
    fill = jnp.full((pad,), N_NODES, jnp.int32)
    src_p = jnp.concatenate([ei[0], fill])
    dst_p = jnp.concatenate([ei[1], fill])
    src16 = src_p.reshape(NS, MSG_NCHUNK, MSG_CBLK, EPB)
    dst16 = dst_p.reshape(NS, MSG_NCHUNK, MSG_CBLK, EPB)
    dst32 = dst_p.reshape(NC * NS, DEG_NBLK, EPB)

    x_p = jnp.zeros((NPAD, IN_C), jnp.float32).at[:N_NODES].set(x)
    ones = jnp.ones((EPB, HALF), jnp.float32)
    zrows = jnp.zeros((RPS, HALF), jnp.float32)

    deg = _deg_kernel()(dst32, ones, zrows).reshape(NC, NPAD, HALF)

    msg = _msg_kernel()
    g1a, g1b = _tc_first(x_p, W1, deg)
    a1a, a1b = msg(src16, dst16, g1a, g1b, zrows)

    g2a, g2b = _tc_mid(a1a, a1b, g1a, g1b, deg, W2, b1.reshape(1, HID_C))
    a2a, a2b = msg(src16, dst16, g2a, g2b, zrows)

    g3a, g3b = _tc_mid(a2a, a2b, g2a, g2b, deg, W3, b2.reshape(1, HID_C))
    a3a, a3b = msg(src16, dst16, g3a, g3b, zrows)

    out = _tc_last(a3a, a3b, g3a, g3b, deg, b3.reshape(1, OUT_C))
    return out[:N_NODES]


# speedup vs baseline: 1.0327x; 1.0014x over previous
import functools

import jax
import jax.numpy as jnp
from jax import lax
from jax.experimental import pallas as pl
from jax.experimental.pallas import tpu as pltpu
from jax.experimental.pallas import tpu_sc as plsc

N_NODES = 10000
N_EDGES = 320000
IN_C = 128
HID_C = 256
OUT_C = 256

NC = 2
NS = 16
EPB = 128

NPAD = 10240
EPAD = 323584
MSG_NBLK = EPAD // NS // EPB
MSG_NCHUNK = 2
MSG_CBLK = MSG_NBLK // MSG_NCHUNK
DEG_NBLK = EPAD // (NC * NS) // EPB
RPS = NPAD // NS
HALF = 128

@functools.cache
def _mesh():
    return plsc.VectorSubcoreMesh(
        core_axis_name="c", subcore_axis_name="s",
        num_cores=NC, num_subcores=NS)


def _deg_body(dst_hbm, ones_hbm, zrows_hbm, deg_hbm, dst_v, ones_v, deg_sh,
              sem):
    c = lax.axis_index("c")
    s = lax.axis_index("s")
    wid = c * NS + s
    pltpu.sync_copy(zrows_hbm, deg_sh.at[pl.ds(s * RPS, RPS)])
    pltpu.sync_copy(ones_hbm, ones_v)
    pltpu.sync_copy(dst_hbm.at[wid], dst_v)
    plsc.subcore_barrier()

    def body(j, carry):
        pltpu.sync_copy(ones_v, deg_sh.at[dst_v.at[j]], add=True)
        return carry

    lax.fori_loop(0, DEG_NBLK, body, 0)
    plsc.subcore_barrier()
    pltpu.sync_copy(deg_sh.at[pl.ds(s * RPS, RPS)],
                    deg_hbm.at[pl.ds(c * NPAD + s * RPS, RPS)])


@functools.cache
def _deg_kernel():
    return pl.kernel(
        _deg_body,
        out_type=jax.ShapeDtypeStruct((NC * NPAD, HALF), jnp.float32),
        mesh=_mesh(),
        scratch_types=[
            pltpu.VMEM((DEG_NBLK, EPB), jnp.int32),
            pltpu.VMEM((EPB, HALF), jnp.float32),
            pltpu.VMEM_SHARED((NPAD, HALF), jnp.float32),
            pltpu.SemaphoreType.DMA,
        ],
    )


def _msg_half(g_hbm, out_hbm, src_hbm, dst_hbm, src_v, dst_v, rows_v,
              acc_sh, sem, s):
    def chunk(k, carry):
        pltpu.sync_copy(src_hbm.at[s, k], src_v)
        pltpu.sync_copy(dst_hbm.at[s, k], dst_v)

        def body(j, c2):
            pltpu.async_copy(g_hbm.at[src_v.at[j]], rows_v, sem).wait()
            pltpu.sync_copy(rows_v, acc_sh.at[dst_v.at[j]], add=True)
            return c2

        lax.fori_loop(0, MSG_CBLK, body, 0)
        return carry

    for k in range(MSG_NCHUNK):
        chunk(k, 0)
    plsc.subcore_barrier()
    pltpu.sync_copy(acc_sh.at[pl.ds(s * RPS, RPS)],
                    out_hbm.at[pl.ds(s * RPS, RPS)])


def _msg_body(src_hbm, dst_hbm, ga_hbm, gb_hbm, zrows_hbm,
              acca_hbm, accb_hbm, src_v, dst_v, rows_v, acc_sh, sem):
    c = lax.axis_index("c")
    s = lax.axis_index("s")
    pltpu.sync_copy(zrows_hbm, acc_sh.at[pl.ds(s * RPS, RPS)])
    plsc.subcore_barrier()

    @pl.when(c == 0)
    def _():
        _msg_half(ga_hbm, acca_hbm, src_hbm, dst_hbm, src_v, dst_v, rows_v,
                  acc_sh, sem, s)

    @pl.when(c == 1)
    def _():
        _msg_half(gb_hbm, accb_hbm, src_hbm, dst_hbm, src_v, dst_v, rows_v,
                  acc_sh, sem, s)


@functools.cache
def _msg_kernel():
    return pl.kernel(
        _msg_body,
        out_type=[jax.ShapeDtypeStruct((NPAD, HALF), jnp.float32),
                  jax.ShapeDtypeStruct((NPAD, HALF), jnp.float32)],
        mesh=_mesh(),
        scratch_types=[
            pltpu.VMEM((MSG_CBLK, EPB), jnp.int32),
            pltpu.VMEM((MSG_CBLK, EPB), jnp.int32),
            pltpu.VMEM((EPB, HALF), jnp.float32),
            pltpu.VMEM_SHARED((NPAD, HALF), jnp.float32),
            pltpu.SemaphoreType.DMA,
        ],
    )


_RB = 1024


def _dinv_of(deg_ref):
    return lax.rsqrt(deg_ref[0, :, :1] + deg_ref[1, :, :1] + 1.0)


def _tc_first_body(x_ref, w_ref, deg_ref, ga_ref, gb_ref):
    dinv = _dinv_of(deg_ref)
    h = jnp.dot(x_ref[...], w_ref[...], preferred_element_type=jnp.float32)
    g = h * dinv
    ga_ref[...] = g[:, :HALF]
    gb_ref[...] = g[:, HALF:]


def _tc_mid_body(acca_ref, accb_ref, ga_ref, gb_ref, deg_ref, w_ref, b_ref,
                 oa_ref, ob_ref):
    dinv = _dinv_of(deg_ref)
    pre = jnp.concatenate(
        [acca_ref[...] + ga_ref[...], accb_ref[...] + gb_ref[...]], axis=1)
    act = jnp.maximum(pre * dinv + b_ref[...], 0.0)
    h = jnp.dot(act, w_ref[...], preferred_element_type=jnp.float32)
    g = h * dinv
    oa_ref[...] = g[:, :HALF]
    ob_ref[...] = g[:, HALF:]


def _tc_last_body(acca_ref, accb_ref, ga_ref, gb_ref, deg_ref, b_ref, o_ref):
    dinv = _dinv_of(deg_ref)
    pre = jnp.concatenate(
        [acca_ref[...] + ga_ref[...], accb_ref[...] + gb_ref[...]], axis=1)
    o_ref[...] = pre * dinv + b_ref[...]


def _half_spec():
    return pl.BlockSpec((_RB, HALF), lambda i: (i, 0))


def _deg_spec():
    return pl.BlockSpec((2, _RB, HALF), lambda i: (0, i, 0))


def _full_spec(cols):
    return pl.BlockSpec((_RB, cols), lambda i: (i, 0))


def _const_spec(r, c):
    return pl.BlockSpec((r, c), lambda i: (0, 0))


def _tc_first(x, w, deg):
    return pl.pallas_call(
        _tc_first_body,
        grid=(NPAD // _RB,),
        in_specs=[_full_spec(IN_C), _const_spec(IN_C, HID_C), _deg_spec()],
        out_specs=[_half_spec(), _half_spec()],
        out_shape=[jax.ShapeDtypeStruct((NPAD, HALF), jnp.float32)] * 2,
    )(x, w, deg)


def _tc_mid(acca, accb, ga, gb, deg, w, b):
    return pl.pallas_call(
        _tc_mid_body,
        grid=(NPAD // _RB,),
        in_specs=[_half_spec(), _half_spec(), _half_spec(), _half_spec(),
                  _deg_spec(), _const_spec(HID_C, HID_C), _const_spec(1, HID_C)],
        out_specs=[_half_spec(), _half_spec()],
        out_shape=[jax.ShapeDtypeStruct((NPAD, HALF), jnp.float32)] * 2,
    )(acca, accb, ga, gb, deg, w, b)


def _tc_last(acca, accb, ga, gb, deg, b):
    return pl.pallas_call(
        _tc_last_body,
        grid=(NPAD // _RB,),
        in_specs=[_half_spec(), _half_spec(), _half_spec(), _half_spec(),
                  _deg_spec(), _const_spec(1, OUT_C)],
        out_specs=_full_spec(OUT_C),
        out_shape=jax.ShapeDtypeStruct((NPAD, OUT_C), jnp.float32),
    )(acca, accb, ga, gb, deg, b)


def kernel(x, edge_index, W1, b1, W2, b2, W3, b3):
    ei = edge_index.astype(jnp.int32)
    pad = EPAD - N_EDGES
    fill = jnp.full((pad,), N_NODES, jnp.int32)
    src_p = jnp.concatenate([ei[0], fill])
    dst_p = jnp.concatenate([ei[1], fill])
    src16 = src_p.reshape(NS, MSG_NCHUNK, MSG_CBLK, EPB)
    dst16 = dst_p.reshape(NS, MSG_NCHUNK, MSG_CBLK, EPB)
    dst32 = dst_p.reshape(NC * NS, DEG_NBLK, EPB)

    x_p = jnp.zeros((NPAD, IN_C), jnp.float32).at[:N_NODES].set(x)
    ones = jnp.ones((EPB, HALF), jnp.float32)
    zrows = jnp.zeros((RPS, HALF), jnp.float32)

    deg = _deg_kernel()(dst32, ones, zrows).reshape(NC, NPAD, HALF)

    msg = _msg_kernel()
    g1a, g1b = _tc_first(x_p, W1, deg)
    a1a, a1b = msg(src16, dst16, g1a, g1b, zrows)

    g2a, g2b = _tc_mid(a1a, a1b, g1a, g1b, deg, W2, b1.reshape(1, HID_C))
    a2a, a2b = msg(src16, dst16, g2a, g2b, zrows)

    g3a, g3b = _tc_mid(a2a, a2b, g2a, g2b, deg, W3, b2.reshape(1, HID_C))
    a3a, a3b = msg(src16, dst16, g3a, g3b, zrows)

    out = _tc_last(a3a, a3b, g3a, g3b, deg, b3.reshape(1, OUT_C))
    return out[:N_NODES]
